# baseline (speedup 1.0000x reference)
import numpy as np
import jax
import jax.numpy as jnp
from jax import lax
from jax.experimental import pallas as pl
from jax.experimental.pallas import tpu as pltpu
from jax.experimental.pallas import tpu_sc as plsc

VOCAB = 100000
D = 768
SEQ = 4096
BATCH = 4
BFLAT = BATCH * SEQ

NC, NS = 2, 16
NW = NC * NS
PPW = SEQ // NW
K = 16
CPB = PPW // K
T = BATCH * CPB
LANES = 16


def _pos_encoding() -> np.ndarray:
    pos = np.arange(SEQ, dtype=np.float64)[:, None]
    i2 = np.arange(0, D, 2, dtype=np.float64)
    enc = np.zeros((SEQ, D), dtype=np.float32)
    enc[:, 0::2] = np.sin(pos / 10000 ** (i2 / D)).astype(np.float32)
    enc[:, 1::2] = np.cos(pos / 10000 ** (i2 / D)).astype(np.float32)
    return enc


_POS = _pos_encoding()


def _body(x_hbm, pos_hbm, emb_hbm, out_hbm,
          idx_v, pos_v, rows_v, psem, g0, g1, o0, o1):
    wid = lax.axis_index("s") * NC + lax.axis_index("c")
    p0 = wid * PPW

    gsem = (g0, g1)
    osem = (o0, o1)

    def gather_src(t):
        b, c = t // CPB, t % CPB
        return emb_hbm.at[idx_v.at[b, pl.ds(c * K, K)]]

    def out_dst(t):
        b, c = t // CPB, t % CPB
        return out_hbm.at[pl.ds(b * SEQ + p0 + c * K, K)]

    pos_cp = pltpu.async_copy(pos_hbm.at[pl.ds(p0, PPW)], pos_v, psem)
    for b in range(BATCH):
        pltpu.sync_copy(x_hbm.at[pl.ds(b * SEQ + p0, PPW)], idx_v.at[b])

    pltpu.async_copy(gather_src(0), rows_v.at[0], g0)
    pltpu.async_copy(gather_src(1), rows_v.at[1], g1)
    pos_cp.wait()

    def step(i, _):
        tt = i * 2
        for bi in range(2):
            t = tt + bi
            c = t % CPB
            pltpu.make_async_copy(gather_src(t), rows_v.at[bi], gsem[bi]).wait()
            def row(r, _):
                pr = c * K + r
                for j in range(D // LANES):
                    v = pos_v[pr, pl.ds(j * LANES, LANES)]
                    plsc.addupdate(rows_v.at[bi, r, pl.ds(j * LANES, LANES)], v)
                return 0
            lax.fori_loop(0, K, row, 0, unroll=False)
            pltpu.async_copy(rows_v.at[bi], out_dst(t), osem[bi])

            @pl.when(t + 2 < T)
            def _():
                pltpu.make_async_copy(rows_v.at[bi], out_dst(t), osem[bi]).wait()
                pltpu.async_copy(gather_src(t + 2), rows_v.at[bi], gsem[bi])
        return 0

    lax.fori_loop(0, T // 2, step, 0, unroll=False)

    pltpu.make_async_copy(rows_v.at[0], out_dst(T - 2), o0).wait()
    pltpu.make_async_copy(rows_v.at[1], out_dst(T - 1), o1).wait()


@jax.jit
def _run(xf, emb):
    mesh = plsc.VectorSubcoreMesh(core_axis_name="c", subcore_axis_name="s",
                                  num_cores=NC, num_subcores=NS)
    pos = jnp.asarray(_POS)
    return pl.kernel(
        _body,
        out_type=jax.ShapeDtypeStruct((BFLAT, D), jnp.float32),
        mesh=mesh,
        scratch_types=[
            pltpu.VMEM((BATCH, PPW), jnp.int32),
            pltpu.VMEM((PPW, D), jnp.float32),
            pltpu.VMEM((2, K, D), jnp.float32),
            pltpu.SemaphoreType.DMA,
            pltpu.SemaphoreType.DMA,
            pltpu.SemaphoreType.DMA,
            pltpu.SemaphoreType.DMA,
            pltpu.SemaphoreType.DMA,
        ],
    )(xf, pos, emb)


def kernel(x, emb):
    xf = x.reshape(-1).astype(jnp.int32)
    out = _run(xf, emb)
    return out.reshape(BATCH, SEQ, D)


# speedup vs baseline: 1.2597x; 1.2597x over previous
import numpy as np
import jax
import jax.numpy as jnp
from jax import lax
from jax.experimental import pallas as pl
from jax.experimental.pallas import tpu as pltpu
from jax.experimental.pallas import tpu_sc as plsc

VOCAB = 100000
D = 768
SEQ = 4096
BATCH = 4
BFLAT = BATCH * SEQ

NC, NS = 2, 16
NW = NC * NS
PPW = SEQ // NW
Q = 32
NQ = PPW // Q
T = BATCH * NQ
NBUF = 3
LANES = 16


def _pos_encoding() -> np.ndarray:
    pos = np.arange(SEQ, dtype=np.float64)[:, None]
    i2 = np.arange(0, D, 2, dtype=np.float64)
    enc = np.zeros((SEQ, D), dtype=np.float32)
    enc[:, 0::2] = np.sin(pos / 10000 ** (i2 / D)).astype(np.float32)
    enc[:, 1::2] = np.cos(pos / 10000 ** (i2 / D)).astype(np.float32)
    return enc


_POS = _pos_encoding()


def _body(x_hbm, pos_hbm, emb_hbm, out_hbm,
          idx_v, pos_v, rows_v, ps0, ps1, g0, g1, g2, o0, o1, o2):
    wid = lax.axis_index("s") * NC + lax.axis_index("c")
    p0 = wid * PPW

    ps = (ps0, ps1)
    gs = (g0, g1, g2)
    os_ = (o0, o1, o2)

    def gather_src(t):
        q, b = t // BATCH, t % BATCH
        return emb_hbm.at[idx_v.at[b, pl.ds(q * Q, Q)]]

    def out_dst(t):
        q, b = t // BATCH, t % BATCH
        return out_hbm.at[pl.ds(b * SEQ + p0 + q * Q, Q)]

    def pos_src(q):
        return pos_hbm.at[pl.ds(p0 + q * Q, Q)]

    for b in range(BATCH):
        pltpu.sync_copy(x_hbm.at[pl.ds(b * SEQ + p0, PPW)], idx_v.at[b])
    pltpu.async_copy(pos_src(0), pos_v.at[0], ps[0])
    for t in range(NBUF):
        pltpu.async_copy(gather_src(t), rows_v.at[t], gs[t])

    for t in range(T):
        q, b = t // BATCH, t % BATCH
        bi = t % NBUF
        pq = q % 2
        if b == 0:
            pltpu.make_async_copy(pos_src(q), pos_v.at[pq], ps[pq]).wait()
            if q + 1 < NQ:
                nq = (q + 1) % 2
                pltpu.async_copy(pos_src(q + 1), pos_v.at[nq], ps[nq])
        if t >= NBUF - 1 and t + 1 < T:
            tn = t + 1 - NBUF
            pltpu.make_async_copy(rows_v.at[tn % NBUF], out_dst(tn),
                                  os_[tn % NBUF]).wait()
            pltpu.async_copy(gather_src(t + 1), rows_v.at[(t + 1) % NBUF],
                             gs[(t + 1) % NBUF])
        pltpu.make_async_copy(gather_src(t), rows_v.at[bi], gs[bi]).wait()

        def row(r, _):
            for j in range(D // LANES):
                v = pos_v[pq, r, pl.ds(j * LANES, LANES)]
                plsc.addupdate(rows_v.at[bi, r, pl.ds(j * LANES, LANES)], v)
            return 0

        lax.fori_loop(0, Q, row, 0, unroll=False)
        pltpu.async_copy(rows_v.at[bi], out_dst(t), os_[bi])

    for t in range(T - NBUF, T):
        pltpu.make_async_copy(rows_v.at[t % NBUF], out_dst(t),
                              os_[t % NBUF]).wait()


@jax.jit
def _run(xf, emb):
    mesh = plsc.VectorSubcoreMesh(core_axis_name="c", subcore_axis_name="s",
                                  num_cores=NC, num_subcores=NS)
    pos = jnp.asarray(_POS)
    return pl.kernel(
        _body,
        out_type=jax.ShapeDtypeStruct((BFLAT, D), jnp.float32),
        mesh=mesh,
        scratch_types=[
            pltpu.VMEM((BATCH, PPW), jnp.int32),
            pltpu.VMEM((2, Q, D), jnp.float32),
            pltpu.VMEM((NBUF, Q, D), jnp.float32),
        ] + [pltpu.SemaphoreType.DMA] * 8,
    )(xf, pos, emb)


def kernel(x, emb):
    xf = x.reshape(-1).astype(jnp.int32)
    out = _run(xf, emb)
    return out.reshape(BATCH, SEQ, D)


# speedup vs baseline: 1.5846x; 1.2579x over previous
import numpy as np
import jax
import jax.numpy as jnp
from jax import lax
from jax.experimental import pallas as pl
from jax.experimental.pallas import tpu as pltpu
from jax.experimental.pallas import tpu_sc as plsc

VOCAB = 100000
D = 768
SEQ = 4096
BATCH = 4
BFLAT = BATCH * SEQ

NC, NS = 2, 16
NW = NC * NS
PPW = SEQ // NW
Q = 16
NQ = PPW // Q
T = BATCH * NQ
NBUF = 4
LANES = 16


def _pos_encoding() -> np.ndarray:
    pos = np.arange(SEQ, dtype=np.float64)[:, None]
    i2 = np.arange(0, D, 2, dtype=np.float64)
    enc = np.zeros((SEQ, D), dtype=np.float32)
    enc[:, 0::2] = np.sin(pos / 10000 ** (i2 / D)).astype(np.float32)
    enc[:, 1::2] = np.cos(pos / 10000 ** (i2 / D)).astype(np.float32)
    return enc


_POS = _pos_encoding()


def _body(x_hbm, pos_hbm, emb_hbm, out_hbm,
          idx_v, pos_v, rows_v, ps0, ps1, g0, g1, g2, g3, o0, o1, o2, o3):
    wid = lax.axis_index("s") * NC + lax.axis_index("c")
    p0 = wid * PPW

    ps = (ps0, ps1)
    gs = (g0, g1, g2, g3)
    os_ = (o0, o1, o2, o3)

    def gather_src(t):
        q, b = t // BATCH, t % BATCH
        return emb_hbm.at[idx_v.at[b, pl.ds(q * Q, Q)]]

    def out_dst(t):
        q, b = t // BATCH, t % BATCH
        return out_hbm.at[pl.ds(b * SEQ + p0 + q * Q, Q)]

    def pos_src(q):
        return pos_hbm.at[pl.ds(p0 + q * Q, Q)]

    for b in range(BATCH):
        pltpu.sync_copy(x_hbm.at[b, pl.ds(p0, PPW)], idx_v.at[b])
    pltpu.async_copy(pos_src(0), pos_v.at[0], ps[0])
    for t in range(NBUF):
        pltpu.async_copy(gather_src(t), rows_v.at[t], gs[t])

    for t in range(T):
        q, b = t // BATCH, t % BATCH
        bi = t % NBUF
        pq = q % 2
        if b == 0:
            pltpu.make_async_copy(pos_src(q), pos_v.at[pq], ps[pq]).wait()
            if q + 1 < NQ:
                nq = (q + 1) % 2
                pltpu.async_copy(pos_src(q + 1), pos_v.at[nq], ps[nq])
        if t >= NBUF - 1 and t + 1 < T:
            tn = t + 1 - NBUF
            pltpu.make_async_copy(rows_v.at[tn % NBUF], out_dst(tn),
                                  os_[tn % NBUF]).wait()
            pltpu.async_copy(gather_src(t + 1), rows_v.at[(t + 1) % NBUF],
                             gs[(t + 1) % NBUF])
        pltpu.make_async_copy(gather_src(t), rows_v.at[bi], gs[bi]).wait()

        def row(r, _):
            for j in range(D // LANES):
                v = pos_v[pq, r, pl.ds(j * LANES, LANES)]
                plsc.addupdate(rows_v.at[bi, r, pl.ds(j * LANES, LANES)], v)
            return 0

        lax.fori_loop(0, Q, row, 0, unroll=False)
        pltpu.async_copy(rows_v.at[bi], out_dst(t), os_[bi])

    for t in range(T - NBUF, T):
        pltpu.make_async_copy(rows_v.at[t % NBUF], out_dst(t),
                              os_[t % NBUF]).wait()


@jax.jit
def _run(x2, emb):
    mesh = plsc.VectorSubcoreMesh(core_axis_name="c", subcore_axis_name="s",
                                  num_cores=NC, num_subcores=NS)
    pos = jnp.asarray(_POS)
    return pl.kernel(
        _body,
        out_type=jax.ShapeDtypeStruct((BFLAT, D), jnp.float32),
        mesh=mesh,
        scratch_types=[
            pltpu.VMEM((BATCH, PPW), jnp.int32),
            pltpu.VMEM((2, Q, D), jnp.float32),
            pltpu.VMEM((NBUF, Q, D), jnp.float32),
        ] + [pltpu.SemaphoreType.DMA] * 10,
    )(x2, pos, emb)


def kernel(x, emb):
    out = _run(x.astype(jnp.int32), emb)
    return out.reshape(BATCH, SEQ, D)
